# baseline (device time: 28518 ns/iter reference)
import jax
import jax.numpy as jnp
from jax import lax
from jax.experimental import pallas as pl
from jax.experimental.pallas import tpu as pltpu

N_DEV = 32
N_CHUNK = 8


def kernel(x, w_mat):
    m_per, k = x.shape
    _, n = w_mat.shape
    n_per = n // N_DEV
    c_cols = n // N_CHUNK
    t_per_c = N_CHUNK * n_per // n * N_DEV // N_CHUNK
    t_per_c = c_cols // n_per

    def body(x_ref, w_hbm, out_ref, w_bufs, tilebuf, dma_sems, send_sems,
             recv_sems):
        my = lax.axis_index("i")
        my_chunk = lax.div(my, t_per_c)
        start = lax.rem(my_chunk + 1, N_CHUNK)

        barrier_sem = pltpu.get_barrier_semaphore()
        for d in range(1, N_DEV):
            pl.semaphore_signal(
                barrier_sem, inc=1,
                device_id=(lax.rem(my + d, N_DEV),),
                device_id_type=pl.DeviceIdType.MESH,
            )

        def w_copy(c, slot):
            off = pl.multiple_of(c * c_cols, c_cols)
            return pltpu.make_async_copy(
                w_hbm.at[:, pl.ds(off, c_cols)],
                w_bufs.at[slot],
                dma_sems.at[slot],
            )

        for j in range(N_CHUNK):
            w_copy(lax.rem(start + j, N_CHUNK), j).start()

        pl.semaphore_wait(barrier_sem, N_DEV - 1)

        for j in range(N_CHUNK):
            c = lax.rem(start + j, N_CHUNK)
            w_copy(c, j).wait()
            y = jnp.maximum(
                jnp.dot(x_ref[...], w_bufs[j],
                        preferred_element_type=jnp.float32),
                0.0,
            )
            for u in range(t_per_c):
                tilebuf[j, u] = y[:, u * n_per:(u + 1) * n_per]
            for u in range(t_per_c):
                t = c * t_per_c + u

                @pl.when(t != my)
                def _():
                    pltpu.make_async_remote_copy(
                        src_ref=tilebuf.at[j, u],
                        dst_ref=out_ref.at[pl.ds(my * m_per, m_per), :],
                        send_sem=send_sems.at[j * t_per_c + u],
                        recv_sem=recv_sems.at[my],
                        device_id=(t,),
                        device_id_type=pl.DeviceIdType.MESH,
                    ).start()

                @pl.when(t == my)
                def _():
                    out_ref[pl.ds(my * m_per, m_per), :] = tilebuf[j, u]

        for d in range(1, N_DEV):
            src = lax.rem(my + d, N_DEV)
            pltpu.make_async_remote_copy(
                src_ref=tilebuf.at[0, 0],
                dst_ref=out_ref.at[pl.ds(src * m_per, m_per), :],
                send_sem=send_sems.at[0],
                recv_sem=recv_sems.at[src],
                device_id=(src,),
                device_id_type=pl.DeviceIdType.MESH,
            ).wait_recv()

        my_u = lax.rem(my, t_per_c)
        for j in range(N_CHUNK):
            for u in range(t_per_c):
                is_own = jnp.logical_and(j == N_CHUNK - 1, u == my_u)

                @pl.when(jnp.logical_not(is_own))
                def _():
                    pltpu.make_async_remote_copy(
                        src_ref=tilebuf.at[j, u],
                        dst_ref=out_ref.at[pl.ds(my * m_per, m_per), :],
                        send_sem=send_sems.at[j * t_per_c + u],
                        recv_sem=recv_sems.at[my],
                        device_id=(my,),
                        device_id_type=pl.DeviceIdType.MESH,
                    ).wait_send()

    return pl.pallas_call(
        body,
        out_shape=jax.ShapeDtypeStruct((N_DEV * m_per, n_per), jnp.float32),
        in_specs=[
            pl.BlockSpec(memory_space=pltpu.VMEM),
            pl.BlockSpec(memory_space=pl.ANY),
        ],
        out_specs=pl.BlockSpec(memory_space=pltpu.VMEM),
        scratch_shapes=[
            pltpu.VMEM((N_CHUNK, k, c_cols), jnp.float32),
            pltpu.VMEM((N_CHUNK, t_per_c, m_per, n_per), jnp.float32),
            pltpu.SemaphoreType.DMA((N_CHUNK,)),
            pltpu.SemaphoreType.DMA((N_DEV,)),
            pltpu.SemaphoreType.DMA((N_DEV,)),
        ],
        compiler_params=pltpu.CompilerParams(collective_id=0),
    )(x, w_mat)
